# baseline (speedup 1.0000x reference)
import functools

import jax
import jax.numpy as jnp
from jax import lax
from jax.experimental import pallas as pl
from jax.experimental.pallas import tpu as pltpu
from jax.experimental.pallas import tpu_sc as plsc

NUM_TEXT = 100000
NUM_POS = 2048
DIM = 64
B = 4096
L = 200

_NC = 2
_NS = 16
_NW = _NC * _NS
_ROWS_PER_W = B // _NW

_CHUNK0 = 128
_CHUNK1 = L - _CHUNK0


def _body(x_hbm, text_hbm, pos_hbm, out_hbm, idx_v, rows0, rows1, pos_v,
          gsem0, gsem1, wsem0, wsem1):
    wid = lax.axis_index("s") * _NC + lax.axis_index("c")
    base = wid * _ROWS_PER_W

    pltpu.sync_copy(pos_hbm.at[pl.ds(0, L)], pos_v)
    pltpu.sync_copy(x_hbm.at[pl.ds(base, _ROWS_PER_W)], idx_v)

    rows = (rows0, rows1)
    gsems = (gsem0, gsem1)
    wsems = (wsem0, wsem1)

    def prefill(p):
        rp = rows[p]

        def fill_row(l, carry2):
            for g in range(DIM // 16):
                sl = pl.ds(g * 16, 16)
                rp[l, sl] = pos_v[l, sl]
            return carry2

        lax.fori_loop(0, L, fill_row, 0)

    def start_gather(i, p):
        pltpu.async_copy(
            text_hbm.at[idx_v.at[i, pl.ds(0, _CHUNK0)]],
            rows[p].at[pl.ds(0, _CHUNK0)],
            gsems[p],
            add=True,
        )
        pltpu.async_copy(
            text_hbm.at[idx_v.at[i, pl.ds(_CHUNK0, _CHUNK1)]],
            rows[p].at[pl.ds(_CHUNK0, _CHUNK1)],
            gsems[p],
            add=True,
        )

    def wait_gather(i, p):
        pltpu.make_async_copy(
            text_hbm.at[idx_v.at[i, pl.ds(0, _CHUNK0)]],
            rows[p].at[pl.ds(0, _CHUNK0)],
            gsems[p],
        ).wait()
        pltpu.make_async_copy(
            text_hbm.at[idx_v.at[i, pl.ds(_CHUNK0, _CHUNK1)]],
            rows[p].at[pl.ds(_CHUNK0, _CHUNK1)],
            gsems[p],
        ).wait()

    def start_write(i, p):
        pltpu.async_copy(
            rows[p],
            out_hbm.at[pl.ds((base + i) * L, L)],
            wsems[p],
        )

    def wait_write(p):
        pltpu.make_async_copy(
            rows[p],
            out_hbm.at[pl.ds(base * L, L)],
            wsems[p],
        ).wait()

    prefill(0)
    start_gather(0, 0)
    prefill(1)
    start_gather(1, 1)
    wait_gather(0, 0)
    start_write(0, 0)

    def step(j, carry):
        for p in range(2):
            i = 2 * j + 2 + p
            other = 1 - p
            wait_write(p)
            prefill(p)
            start_gather(i, p)
            wait_gather(i - 1, other)
            start_write(i - 1, other)
        return carry

    lax.fori_loop(0, (_ROWS_PER_W - 4) // 2, step, 0)

    wait_write(0)
    prefill(0)
    start_gather(_ROWS_PER_W - 2, 0)
    wait_gather(_ROWS_PER_W - 3, 1)
    start_write(_ROWS_PER_W - 3, 1)
    wait_write(1)
    prefill(1)
    start_gather(_ROWS_PER_W - 1, 1)
    wait_gather(_ROWS_PER_W - 2, 0)
    start_write(_ROWS_PER_W - 2, 0)
    wait_gather(_ROWS_PER_W - 1, 1)
    start_write(_ROWS_PER_W - 1, 1)
    wait_write(0)
    wait_write(1)


@jax.jit
def kernel(x, text_table, pos_table):
    mesh = plsc.VectorSubcoreMesh(core_axis_name="c", subcore_axis_name="s")
    run = functools.partial(
        pl.kernel,
        out_type=jax.ShapeDtypeStruct((B * L, DIM), jnp.float32),
        mesh=mesh,
        scratch_types=[
            pltpu.VMEM((_ROWS_PER_W, L), jnp.int32),
            pltpu.VMEM((L, DIM), jnp.float32),
            pltpu.VMEM((L, DIM), jnp.float32),
            pltpu.VMEM((L, DIM), jnp.float32),
            pltpu.SemaphoreType.DMA,
            pltpu.SemaphoreType.DMA,
            pltpu.SemaphoreType.DMA,
            pltpu.SemaphoreType.DMA,
        ],
        compiler_params=pltpu.CompilerParams(use_tc_tiling_on_sc=False),
    )(_body)
    out = run(x.astype(jnp.int32), text_table, pos_table)
    return jnp.reshape(out, (B, L, DIM))


# speedup vs baseline: 1.0176x; 1.0176x over previous
import functools

import jax
import jax.numpy as jnp
from jax import lax
from jax.experimental import pallas as pl
from jax.experimental.pallas import tpu as pltpu
from jax.experimental.pallas import tpu_sc as plsc

NUM_TEXT = 100000
NUM_POS = 2048
DIM = 64
B = 4096
L = 200

_NC = 2
_NS = 16
_NW = _NC * _NS
_ROWS_PER_W = B // _NW

_CHUNK0 = 128
_CHUNK1 = L - _CHUNK0


def _body(x_hbm, text_hbm, pos_hbm, out_hbm, idx_v, rows0, rows1, pos_v,
          gsem0, gsem1, wsem0, wsem1):
    wid = lax.axis_index("s") * _NC + lax.axis_index("c")
    base = wid * _ROWS_PER_W

    pltpu.sync_copy(pos_hbm.at[pl.ds(0, L)], pos_v)
    pltpu.sync_copy(x_hbm.at[pl.ds(base, _ROWS_PER_W)], idx_v)

    rows = (rows0, rows1)
    gsems = (gsem0, gsem1)
    wsems = (wsem0, wsem1)

    def prefill(p):
        rp = rows[p]

        def fill_row(l4, carry2):
            for u in range(4):
                l = l4 * 4 + u
                for g in range(DIM // 16):
                    sl = pl.ds(g * 16, 16)
                    rp[l, sl] = pos_v[l, sl]
            return carry2

        lax.fori_loop(0, L // 4, fill_row, 0)

    def start_gather(i, p):
        pltpu.async_copy(
            text_hbm.at[idx_v.at[i, pl.ds(0, _CHUNK0)]],
            rows[p].at[pl.ds(0, _CHUNK0)],
            gsems[p],
            add=True,
        )
        pltpu.async_copy(
            text_hbm.at[idx_v.at[i, pl.ds(_CHUNK0, _CHUNK1)]],
            rows[p].at[pl.ds(_CHUNK0, _CHUNK1)],
            gsems[p],
            add=True,
        )

    def wait_gather(i, p):
        pltpu.make_async_copy(
            text_hbm.at[idx_v.at[i, pl.ds(0, _CHUNK0)]],
            rows[p].at[pl.ds(0, _CHUNK0)],
            gsems[p],
        ).wait()
        pltpu.make_async_copy(
            text_hbm.at[idx_v.at[i, pl.ds(_CHUNK0, _CHUNK1)]],
            rows[p].at[pl.ds(_CHUNK0, _CHUNK1)],
            gsems[p],
        ).wait()

    def start_write(i, p):
        pltpu.async_copy(
            rows[p],
            out_hbm.at[pl.ds((base + i) * L, L)],
            wsems[p],
        )

    def wait_write(p):
        pltpu.make_async_copy(
            rows[p],
            out_hbm.at[pl.ds(base * L, L)],
            wsems[p],
        ).wait()

    prefill(0)
    start_gather(0, 0)
    prefill(1)
    start_gather(1, 1)
    wait_gather(0, 0)
    start_write(0, 0)

    def step(j, carry):
        for p in range(2):
            i = 2 * j + 2 + p
            other = 1 - p
            wait_write(p)
            prefill(p)
            start_gather(i, p)
            wait_gather(i - 1, other)
            start_write(i - 1, other)
        return carry

    lax.fori_loop(0, (_ROWS_PER_W - 4) // 2, step, 0)

    wait_write(0)
    prefill(0)
    start_gather(_ROWS_PER_W - 2, 0)
    wait_gather(_ROWS_PER_W - 3, 1)
    start_write(_ROWS_PER_W - 3, 1)
    wait_write(1)
    prefill(1)
    start_gather(_ROWS_PER_W - 1, 1)
    wait_gather(_ROWS_PER_W - 2, 0)
    start_write(_ROWS_PER_W - 2, 0)
    wait_gather(_ROWS_PER_W - 1, 1)
    start_write(_ROWS_PER_W - 1, 1)
    wait_write(0)
    wait_write(1)


@jax.jit
def kernel(x, text_table, pos_table):
    mesh = plsc.VectorSubcoreMesh(core_axis_name="c", subcore_axis_name="s")
    run = functools.partial(
        pl.kernel,
        out_type=jax.ShapeDtypeStruct((B * L, DIM), jnp.float32),
        mesh=mesh,
        scratch_types=[
            pltpu.VMEM((_ROWS_PER_W, L), jnp.int32),
            pltpu.VMEM((L, DIM), jnp.float32),
            pltpu.VMEM((L, DIM), jnp.float32),
            pltpu.VMEM((L, DIM), jnp.float32),
            pltpu.SemaphoreType.DMA,
            pltpu.SemaphoreType.DMA,
            pltpu.SemaphoreType.DMA,
            pltpu.SemaphoreType.DMA,
        ],
        compiler_params=pltpu.CompilerParams(use_tc_tiling_on_sc=False),
    )(_body)
    out = run(x.astype(jnp.int32), text_table, pos_table)
    return jnp.reshape(out, (B, L, DIM))
